# baseline (speedup 1.0000x reference)
import functools

import jax
import jax.numpy as jnp
from jax import lax
from jax.experimental import pallas as pl
from jax.experimental.pallas import tpu as pltpu
from jax.experimental.pallas import tpu_sc as plsc

NX = 24
NY = 24
P = NX * NY
E = 768
B = 64
LANES = 16
UNROLL = 8


def _pos_sc_kernel(row_hbm, col_hbm, pos_hbm, row_v, col_v, out_v):
    wid = lax.axis_index("s") * 2 + lax.axis_index("c")

    @pl.when(wid < NX)
    def _():
        pltpu.sync_copy(row_hbm.at[wid, :], row_v)
        pltpu.sync_copy(col_hbm, col_v)

        def one_row(c, _):
            @plsc.parallel_loop(0, E, LANES, unroll=UNROLL)
            def chunk(j):
                out_v[c, pl.ds(j, LANES)] = (
                    row_v[pl.ds(j, LANES)] + col_v[c, pl.ds(j, LANES)]
                )

            return 0

        lax.fori_loop(0, NY, one_row, 0)
        pltpu.sync_copy(out_v, pos_hbm.at[pl.ds(NY * wid, NY), :])


def _add_tc_kernel(x_ref, p_ref, o_ref):
    o_ref[...] = x_ref[...] + p_ref[...][None, :, :]


@jax.jit
def _run(x, rows, cols):
    mesh = plsc.VectorSubcoreMesh(core_axis_name="c", subcore_axis_name="s")
    pos = pl.kernel(
        _pos_sc_kernel,
        mesh=mesh,
        out_type=jax.ShapeDtypeStruct((P, E), jnp.float32),
        scratch_types=[
            pltpu.VMEM((E,), jnp.float32),
            pltpu.VMEM((NY, E), jnp.float32),
            pltpu.VMEM((NY, E), jnp.float32),
        ],
    )(rows, cols)

    return pl.pallas_call(
        _add_tc_kernel,
        grid=(4, B),
        in_specs=[
            pl.BlockSpec((1, P // 4, E), lambda q, b: (b, q, 0)),
            pl.BlockSpec((P // 4, E), lambda q, b: (q, 0)),
        ],
        out_specs=pl.BlockSpec((1, P // 4, E), lambda q, b: (b, q, 0)),
        out_shape=jax.ShapeDtypeStruct((B, P, E), jnp.float32),
        compiler_params=pltpu.CompilerParams(
            dimension_semantics=("arbitrary", "arbitrary"),
        ),
    )(x, pos)


def kernel(x, row_embed, col_embed):
    return _run(x, row_embed, col_embed)


# speedup vs baseline: 2.0783x; 2.0783x over previous
import functools

import jax
import jax.numpy as jnp
from jax import lax
from jax.experimental import pallas as pl
from jax.experimental.pallas import tpu as pltpu
from jax.experimental.pallas import tpu_sc as plsc

NX = 24
NY = 24
P = NX * NY
E = 768
B = 64
LANES = 16
UNROLL = 8


def _pos_sc_kernel(row_hbm, col_hbm, pos_hbm, row_v, col_v, out_v):
    wid = lax.axis_index("s") * 2 + lax.axis_index("c")

    @pl.when(wid < NX)
    def _():
        pltpu.sync_copy(row_hbm.at[wid, :], row_v)
        pltpu.sync_copy(col_hbm, col_v)

        def one_row(c, _):
            @plsc.parallel_loop(0, E, LANES, unroll=UNROLL)
            def chunk(j):
                out_v[c, pl.ds(j, LANES)] = (
                    row_v[pl.ds(j, LANES)] + col_v[c, pl.ds(j, LANES)]
                )

            return 0

        lax.fori_loop(0, NY, one_row, 0)
        pltpu.sync_copy(out_v, pos_hbm.at[pl.ds(NY * wid, NY), :])


def _add_tc_kernel(x_ref, p_ref, o_ref):
    o_ref[...] = x_ref[...] + p_ref[...][None, :, :]


@jax.jit
def _run(x, rows, cols):
    mesh = plsc.VectorSubcoreMesh(core_axis_name="c", subcore_axis_name="s")
    pos = pl.kernel(
        _pos_sc_kernel,
        mesh=mesh,
        out_type=jax.ShapeDtypeStruct((P, E), jnp.float32),
        scratch_types=[
            pltpu.VMEM((E,), jnp.float32),
            pltpu.VMEM((NY, E), jnp.float32),
            pltpu.VMEM((NY, E), jnp.float32),
        ],
    )(rows, cols)

    return pl.pallas_call(
        _add_tc_kernel,
        grid=(B // 2,),
        in_specs=[
            pl.BlockSpec((2, P, E), lambda b: (b, 0, 0)),
            pl.BlockSpec((P, E), lambda b: (0, 0)),
        ],
        out_specs=pl.BlockSpec((2, P, E), lambda b: (b, 0, 0)),
        out_shape=jax.ShapeDtypeStruct((B, P, E), jnp.float32),
        compiler_params=pltpu.CompilerParams(
            dimension_semantics=("arbitrary",),
        ),
    )(x, pos)


def kernel(x, row_embed, col_embed):
    return _run(x, row_embed, col_embed)


# speedup vs baseline: 2.1234x; 1.0217x over previous
import functools

import jax
import jax.numpy as jnp
from jax import lax
from jax.experimental import pallas as pl
from jax.experimental.pallas import tpu as pltpu
from jax.experimental.pallas import tpu_sc as plsc

NX = 24
NY = 24
P = NX * NY
E = 768
B = 64
LANES = 16
UNROLL = 8


def _pos_sc_kernel(row_hbm, col_hbm, pos_hbm, row_v, col_v, out_v):
    wid = lax.axis_index("s") * 2 + lax.axis_index("c")

    @pl.when(wid < NX)
    def _():
        pltpu.sync_copy(row_hbm.at[wid, :], row_v)
        pltpu.sync_copy(col_hbm, col_v)

        def one_row(c, _):
            @plsc.parallel_loop(0, E, LANES, unroll=UNROLL)
            def chunk(j):
                out_v[c, pl.ds(j, LANES)] = (
                    row_v[pl.ds(j, LANES)] + col_v[c, pl.ds(j, LANES)]
                )

            return 0

        lax.fori_loop(0, NY, one_row, 0)
        pltpu.sync_copy(out_v, pos_hbm.at[pl.ds(NY * wid, NY), :])


def _add_tc_kernel(x_ref, p_ref, o_ref):
    o_ref[...] = x_ref[...] + p_ref[...][None, :, :]


@jax.jit
def _run(x, rows, cols):
    mesh = plsc.VectorSubcoreMesh(core_axis_name="c", subcore_axis_name="s")
    pos = pl.kernel(
        _pos_sc_kernel,
        mesh=mesh,
        out_type=jax.ShapeDtypeStruct((P, E), jnp.float32),
        scratch_types=[
            pltpu.VMEM((E,), jnp.float32),
            pltpu.VMEM((NY, E), jnp.float32),
            pltpu.VMEM((NY, E), jnp.float32),
        ],
    )(rows, cols)

    return pl.pallas_call(
        _add_tc_kernel,
        grid=(B // 4,),
        in_specs=[
            pl.BlockSpec((4, P, E), lambda b: (b, 0, 0)),
            pl.BlockSpec((P, E), lambda b: (0, 0)),
        ],
        out_specs=pl.BlockSpec((4, P, E), lambda b: (b, 0, 0)),
        out_shape=jax.ShapeDtypeStruct((B, P, E), jnp.float32),
        compiler_params=pltpu.CompilerParams(
            dimension_semantics=("arbitrary",),
        ),
    )(x, pos)


def kernel(x, row_embed, col_embed):
    return _run(x, row_embed, col_embed)


# speedup vs baseline: 2.1490x; 1.0121x over previous
import functools

import jax
import jax.numpy as jnp
from jax import lax
from jax.experimental import pallas as pl
from jax.experimental.pallas import tpu as pltpu
from jax.experimental.pallas import tpu_sc as plsc

NX = 24
NY = 24
P = NX * NY
E = 768
B = 64
LANES = 16
UNROLL = 8


def _pos_sc_kernel(row_hbm, col_hbm, pos_hbm, row_v, col_v, out_v):
    wid = lax.axis_index("s") * 2 + lax.axis_index("c")

    @pl.when(wid < NX)
    def _():
        pltpu.sync_copy(row_hbm.at[wid, :], row_v)
        pltpu.sync_copy(col_hbm, col_v)

        def one_row(c, _):
            @plsc.parallel_loop(0, E, LANES, unroll=UNROLL)
            def chunk(j):
                out_v[c, pl.ds(j, LANES)] = (
                    row_v[pl.ds(j, LANES)] + col_v[c, pl.ds(j, LANES)]
                )

            return 0

        lax.fori_loop(0, NY, one_row, 0)
        pltpu.sync_copy(out_v, pos_hbm.at[pl.ds(NY * wid, NY), :])


def _add_tc_kernel(x_ref, p_ref, o_ref):
    o_ref[...] = x_ref[...] + p_ref[...][None, :, :]


@jax.jit
def _run(x, rows, cols):
    mesh = plsc.VectorSubcoreMesh(core_axis_name="c", subcore_axis_name="s")
    pos = pl.kernel(
        _pos_sc_kernel,
        mesh=mesh,
        out_type=jax.ShapeDtypeStruct((P, E), jnp.float32),
        scratch_types=[
            pltpu.VMEM((E,), jnp.float32),
            pltpu.VMEM((NY, E), jnp.float32),
            pltpu.VMEM((NY, E), jnp.float32),
        ],
    )(rows, cols)

    return pl.pallas_call(
        _add_tc_kernel,
        grid=(B // 8,),
        in_specs=[
            pl.BlockSpec((8, P, E), lambda b: (b, 0, 0)),
            pl.BlockSpec((P, E), lambda b: (0, 0)),
        ],
        out_specs=pl.BlockSpec((8, P, E), lambda b: (b, 0, 0)),
        out_shape=jax.ShapeDtypeStruct((B, P, E), jnp.float32),
        compiler_params=pltpu.CompilerParams(
            dimension_semantics=("arbitrary",),
        ),
    )(x, pos)


def kernel(x, row_embed, col_embed):
    return _run(x, row_embed, col_embed)


# speedup vs baseline: 2.1655x; 1.0077x over previous
import functools

import jax
import jax.numpy as jnp
from jax import lax
from jax.experimental import pallas as pl
from jax.experimental.pallas import tpu as pltpu
from jax.experimental.pallas import tpu_sc as plsc

NX = 24
NY = 24
P = NX * NY
E = 768
B = 64
LANES = 16
UNROLL = 8


def _pos_sc_kernel(row_hbm, col_hbm, pos_hbm, row_v, col_v, out_v, sem0, sem1):
    wid = lax.axis_index("s") * 2 + lax.axis_index("c")

    @pl.when(wid < NX)
    def _():
        row_cp = pltpu.make_async_copy(row_hbm.at[wid, :], row_v, sem0)
        col_cp = pltpu.make_async_copy(col_hbm, col_v, sem1)
        row_cp.start()
        col_cp.start()
        row_cp.wait()
        col_cp.wait()

        def one_row(c, _):
            @plsc.parallel_loop(0, E, LANES, unroll=UNROLL)
            def chunk(j):
                out_v[c, pl.ds(j, LANES)] = (
                    row_v[pl.ds(j, LANES)] + col_v[c, pl.ds(j, LANES)]
                )

            return 0

        lax.fori_loop(0, NY, one_row, 0)
        pltpu.sync_copy(out_v, pos_hbm.at[pl.ds(NY * wid, NY), :])


def _add_tc_kernel(x_ref, p_ref, o_ref):
    o_ref[...] = x_ref[...] + p_ref[...][None, :, :]


@jax.jit
def _run(x, rows, cols):
    mesh = plsc.VectorSubcoreMesh(core_axis_name="c", subcore_axis_name="s")
    pos = pl.kernel(
        _pos_sc_kernel,
        mesh=mesh,
        out_type=jax.ShapeDtypeStruct((P, E), jnp.float32),
        scratch_types=[
            pltpu.VMEM((E,), jnp.float32),
            pltpu.VMEM((NY, E), jnp.float32),
            pltpu.VMEM((NY, E), jnp.float32),
            pltpu.SemaphoreType.DMA,
            pltpu.SemaphoreType.DMA,
        ],
    )(rows, cols)

    return pl.pallas_call(
        _add_tc_kernel,
        grid=(B // 8,),
        in_specs=[
            pl.BlockSpec((8, P, E), lambda b: (b, 0, 0)),
            pl.BlockSpec((P, E), lambda b: (0, 0)),
        ],
        out_specs=pl.BlockSpec((8, P, E), lambda b: (b, 0, 0)),
        out_shape=jax.ShapeDtypeStruct((B, P, E), jnp.float32),
        compiler_params=pltpu.CompilerParams(
            dimension_semantics=("arbitrary",),
        ),
    )(x, pos)


def kernel(x, row_embed, col_embed):
    return _run(x, row_embed, col_embed)
